# baseline (speedup 1.0000x reference)
import functools

import jax
import jax.numpy as jnp
from jax import lax
from jax.experimental import pallas as pl
from jax.experimental.pallas import tpu as pltpu
from jax.experimental.pallas import tpu_sc as plsc

_C = 2048
_SCALE = float(_C) ** 0.5
_NC, _NS, _L = 2, 16, 16
_NW = _NC * _NS
_CH = 8
_NVEC = _C // _L
_SC_ROWS = 6144


def _tc_body(step_ref, x_ref, emb_ref, o_ref):
    del step_ref
    o_ref[...] = x_ref[...] + emb_ref[0] * _SCALE


def _sc_body(x_hbm, step_hbm, emb_hbm, out_hbm,
             idx_v, gath_v, in0, in1, ou0, ou1,
             s_gath, s_in0, s_in1, s_out0, s_out1):
    wid = lax.axis_index("s") * _NC + lax.axis_index("c")
    n_rows = out_hbm.shape[0]
    rows_pw = n_rows // _NW
    n_chunks = rows_pw // _CH
    src0 = x_hbm.shape[0] - n_rows
    base = wid * rows_pw

    pltpu.sync_copy(step_hbm, idx_v)
    pltpu.async_copy(emb_hbm.at[idx_v], gath_v, s_gath).wait()
    for k in range(_NVEC):
        sl = pl.ds(k * _L, _L)
        gath_v[0, sl] = gath_v[0, sl] * _SCALE

    in_bufs = (in0, in1)
    out_bufs = (ou0, ou1)
    s_ins = (s_in0, s_in1)
    s_outs = (s_out0, s_out1)

    def start_in(g, b):
        pltpu.async_copy(x_hbm.at[pl.ds(src0 + base + g * _CH, _CH)],
                         in_bufs[b], s_ins[b])

    def wait_in(b):
        pltpu.make_async_copy(x_hbm.at[pl.ds(0, _CH)], in_bufs[b],
                              s_ins[b]).wait()

    def start_out(g, b):
        pltpu.async_copy(out_bufs[b], out_hbm.at[pl.ds(base + g * _CH, _CH)],
                         s_outs[b])

    def wait_out(b):
        pltpu.make_async_copy(out_bufs[b], out_hbm.at[pl.ds(0, _CH)],
                              s_outs[b]).wait()

    start_in(0, 0)
    start_in(1, 1)

    @pl.loop(0, n_chunks, step=2)
    def _chunks(g0):
        for b in range(2):
            g = g0 + b
            wait_in(b)

            @pl.when(g >= 2)
            def _():
                wait_out(b)

            for k in range(_NVEC):
                sl = pl.ds(k * _L, _L)
                sig = gath_v[0, sl]
                for r in range(_CH):
                    out_bufs[b][r, sl] = in_bufs[b][r, sl] + sig

            start_out(g, b)

            @pl.when(g + 2 < n_chunks)
            def _():
                start_in(g + 2, b)

    wait_out(0)
    wait_out(1)


def kernel(x_layer, step, step_embedding):
    B, S, C = x_layer.shape
    N = B * S
    n_tc = N - _SC_ROWS
    x2 = x_layer.reshape(N, C)
    emb2 = step_embedding.reshape(-1, C)
    emb3 = step_embedding.reshape(-1, 1, C)
    step_i32 = jnp.asarray(step, jnp.int32)
    step_tc = jnp.atleast_1d(step_i32)
    step_sc = jnp.full((8,), step_i32, dtype=jnp.int32)

    mesh = plsc.VectorSubcoreMesh(core_axis_name="c", subcore_axis_name="s",
                                  num_cores=_NC, num_subcores=_NS)
    sc = functools.partial(
        pl.kernel,
        out_type=jax.ShapeDtypeStruct((_SC_ROWS, C), jnp.float32),
        mesh=mesh,
        scratch_types=[
            pltpu.VMEM((8,), jnp.int32),
            pltpu.VMEM((8, C), jnp.float32),
            pltpu.VMEM((_CH, C), jnp.float32),
            pltpu.VMEM((_CH, C), jnp.float32),
            pltpu.VMEM((_CH, C), jnp.float32),
            pltpu.VMEM((_CH, C), jnp.float32),
            pltpu.SemaphoreType.DMA,
            pltpu.SemaphoreType.DMA,
            pltpu.SemaphoreType.DMA,
            pltpu.SemaphoreType.DMA,
            pltpu.SemaphoreType.DMA,
        ],
    )(_sc_body)
    out_sc = sc(x2, step_sc, emb2)

    rows = 1024
    out_tc = pl.pallas_call(
        _tc_body,
        grid_spec=pltpu.PrefetchScalarGridSpec(
            num_scalar_prefetch=1,
            grid=(n_tc // rows,),
            in_specs=[
                pl.BlockSpec((rows, C), lambda i, s: (i, 0)),
                pl.BlockSpec((1, 1, C), lambda i, s: (s[0], 0, 0)),
            ],
            out_specs=pl.BlockSpec((rows, C), lambda i, s: (i, 0)),
        ),
        out_shape=jax.ShapeDtypeStruct((n_tc, C), jnp.float32),
        compiler_params=pltpu.CompilerParams(
            dimension_semantics=("arbitrary",),
        ),
    )(step_tc, x2, emb3)

    out = jnp.concatenate([out_tc, out_sc], axis=0)
    return out.reshape(B, S, C)


# speedup vs baseline: 2.4949x; 2.4949x over previous
import jax
import jax.numpy as jnp
from jax.experimental import pallas as pl
from jax.experimental.pallas import tpu as pltpu

_NUM_CHANNELS = 2048
_SCALE = float(_NUM_CHANNELS) ** 0.5


def _body(step_ref, x_ref, emb_ref, o_ref):
    del step_ref
    o_ref[...] = x_ref[...] + emb_ref[0] * _SCALE


def kernel(x_layer, step, step_embedding):
    B, S, C = x_layer.shape
    N = B * S
    x2 = x_layer.reshape(N, C)
    emb = step_embedding.reshape(-1, 1, C)
    step_arr = jnp.atleast_1d(jnp.asarray(step, jnp.int32))

    rows = 1024
    grid = (N // rows,)

    out = pl.pallas_call(
        _body,
        grid_spec=pltpu.PrefetchScalarGridSpec(
            num_scalar_prefetch=1,
            grid=grid,
            in_specs=[
                pl.BlockSpec((rows, C), lambda i, s: (i, 0)),
                pl.BlockSpec((1, 1, C), lambda i, s: (s[0], 0, 0)),
            ],
            out_specs=pl.BlockSpec((rows, C), lambda i, s: (i, 0)),
        ),
        out_shape=jax.ShapeDtypeStruct((N, C), x_layer.dtype),
        compiler_params=pltpu.CompilerParams(
            dimension_semantics=("arbitrary",),
        ),
    )(step_arr, x2, emb)
    return out.reshape(B, S, C)
